# baseline (speedup 1.0000x reference)
import functools

import jax
import jax.numpy as jnp
from jax.experimental import pallas as pl
from jax.experimental.pallas import tpu as pltpu


def _pick_bm(n: int) -> int:
    best = 8
    for cand in range(8, 257, 8):
        if n % (2 * cand) == 0:
            best = cand
    return best


def _body(x_ref, adj_a, adj_b, w_ref, b_ref, wg_ref, bg_ref, out_ref,
          hidden_ref, *, bm):
    i = pl.program_id(0)

    @pl.when(i == 0)
    def _():
        hidden_ref[...] = (jax.lax.dot_general(
            x_ref[...], w_ref[...],
            dimension_numbers=(((1,), (1,)), ((), ())),
            preferred_element_type=jnp.float32,
        ) + b_ref[...]).astype(jnp.bfloat16)

    def half(adj_ref, row0):
        support = jnp.dot(adj_ref[...].astype(jnp.bfloat16), hidden_ref[...],
                          preferred_element_type=jnp.float32)
        xb = x_ref[pl.ds(row0, bm), :]
        gate = jax.nn.sigmoid(
            jnp.dot(xb, wg_ref[...], preferred_element_type=jnp.float32)
            + bg_ref[...])
        return gate * jnp.maximum(support, 0.0) + (1.0 - gate) * xb

    out_ref[:bm, :] = half(adj_a, i * 2 * bm)
    out_ref[bm:, :] = half(adj_b, i * 2 * bm + bm)


def kernel(x, adj, W, b, W_gate, b_gate):
    n, d = x.shape
    bm = _pick_bm(n)
    grid = (n // (2 * bm),)
    body = functools.partial(_body, bm=bm)
    return pl.pallas_call(
        body,
        grid=grid,
        in_specs=[
            pl.BlockSpec((n, d), lambda i: (0, 0)),
            pl.BlockSpec((bm, n), lambda i: (2 * i, 0)),
            pl.BlockSpec((bm, n), lambda i: (2 * i + 1, 0)),
            pl.BlockSpec((d, d), lambda i: (0, 0)),
            pl.BlockSpec((1, d), lambda i: (0, 0)),
            pl.BlockSpec((d, d), lambda i: (0, 0)),
            pl.BlockSpec((1, d), lambda i: (0, 0)),
        ],
        out_specs=pl.BlockSpec((2 * bm, d), lambda i: (i, 0)),
        out_shape=jax.ShapeDtypeStruct((n, d), jnp.float32),
        scratch_shapes=[pltpu.VMEM((n, d), jnp.bfloat16)],
        compiler_params=pltpu.CompilerParams(
            dimension_semantics=("arbitrary",),
        ),
    )(x, adj, adj, W, b.reshape(1, d), W_gate, b_gate.reshape(1, d))


# speedup vs baseline: 1.0269x; 1.0269x over previous
import functools

import jax
import jax.numpy as jnp
from jax.experimental import pallas as pl
from jax.experimental.pallas import tpu as pltpu


def _pick_bm(n: int) -> int:
    best = 8
    for cand in range(8, 513, 8):
        if n % cand == 0:
            best = cand
    return best


def _body(x_ref, adj_ref, w_ref, b_ref, wg_ref, bg_ref, out_ref, hidden_ref,
          *, bm):
    i = pl.program_id(0)

    @pl.when(i == 0)
    def _():
        hidden_ref[...] = (jax.lax.dot_general(
            x_ref[...], w_ref[...],
            dimension_numbers=(((1,), (1,)), ((), ())),
            preferred_element_type=jnp.float32,
        ) + b_ref[...])

    support = jnp.dot(adj_ref[...], hidden_ref[...],
                      precision=jax.lax.Precision.DEFAULT,
                      preferred_element_type=jnp.float32)
    xb = x_ref[pl.ds(i * bm, bm), :]
    gate = jax.nn.sigmoid(
        jnp.dot(xb, wg_ref[...], preferred_element_type=jnp.float32)
        + bg_ref[...])
    out_ref[...] = gate * jnp.maximum(support, 0.0) + (1.0 - gate) * xb


def kernel(x, adj, W, b, W_gate, b_gate):
    n, d = x.shape
    bm = _pick_bm(n)
    grid = (n // bm,)
    body = functools.partial(_body, bm=bm)
    return pl.pallas_call(
        body,
        grid=grid,
        in_specs=[
            pl.BlockSpec((n, d), lambda i: (0, 0)),
            pl.BlockSpec((bm, n), lambda i: (i, 0)),
            pl.BlockSpec((d, d), lambda i: (0, 0)),
            pl.BlockSpec((1, d), lambda i: (0, 0)),
            pl.BlockSpec((d, d), lambda i: (0, 0)),
            pl.BlockSpec((1, d), lambda i: (0, 0)),
        ],
        out_specs=pl.BlockSpec((bm, d), lambda i: (i, 0)),
        out_shape=jax.ShapeDtypeStruct((n, d), jnp.float32),
        scratch_shapes=[pltpu.VMEM((n, d), jnp.float32)],
        compiler_params=pltpu.CompilerParams(
            dimension_semantics=("arbitrary",),
        ),
    )(x, adj, W, b.reshape(1, d), W_gate, b_gate.reshape(1, d))
